# baseline (speedup 1.0000x reference)
import functools

import numpy as np

import jax
import jax.numpy as jnp
from jax import lax
from jax.experimental import pallas as pl
from jax.experimental.pallas import tpu as pltpu
from jax.experimental.pallas import tpu_sc as plsc

_B, _N, _D, _S = 2, 50000, 40, 4
_DP = 64
_BN = _B * _N
_Q = _S * _BN
_R = 2 * _Q

_NC, _NS = 2, 16
_NW = _NC * _NS
_RW = _R // _NW
_SUB = 128
_K = 16
_MB = _SUB * _K
_NFULL = _RW // _MB
_TAIL = _RW - _NFULL * _MB


_TF_ROTS = ((13, 15, 26, 6), (17, 29, 16, 24),
            (13, 15, 26, 6), (17, 29, 16, 24), (13, 15, 26, 6))


def _np_threefry2x32(k0, k1, x0, x1):
    with np.errstate(over="ignore"):
        ks = [np.uint32(k0), np.uint32(k1),
              np.uint32(k0) ^ np.uint32(k1) ^ np.uint32(0x1BD11BDA)]
        x = [np.uint32(x0) + ks[0], np.uint32(x1) + ks[1]]
        for i, rots in enumerate(_TF_ROTS):
            for r in rots:
                x[0] = (x[0] + x[1]).astype(np.uint32)
                x[1] = x[0] ^ ((x[1] << np.uint32(r))
                               | (x[1] >> np.uint32(32 - r)))
            x[0] = (x[0] + ks[(i + 1) % 3]).astype(np.uint32)
            x[1] = (x[1] + ks[(i + 2) % 3] + np.uint32(i + 1)).astype(
                np.uint32)
        return x[0], x[1]


def _np_split_keys(seed):
    b1, b2 = _np_threefry2x32(0, seed,
                              np.uint32([0, 0]), np.uint32([0, 1]))
    return (int(b1[0]), int(b2[0])), (int(b1[1]), int(b2[1]))


def _tf_rounds(k0, k1, x0, x1):
    ks = (jnp.uint32(k0), jnp.uint32(k1),
          jnp.uint32(np.uint32(k0) ^ np.uint32(k1) ^ np.uint32(0x1BD11BDA)))
    x0 = x0 + ks[0]
    x1 = x1 + ks[1]
    for i, rots in enumerate(_TF_ROTS):
        for r in rots:
            x0 = x0 + x1
            x1 = x0 ^ ((x1 << jnp.uint32(r)) | (x1 >> jnp.uint32(32 - r)))
        x0 = x0 + ks[(i + 1) % 3]
        x1 = x1 + ks[(i + 2) % 3] + jnp.uint32(i + 1)
    return x0, x1


_IDXR = _Q // 128


def _idx_body(outj_ref, outk_ref):
    rows = lax.broadcasted_iota(jnp.uint32, (_IDXR, 128), 0)
    lanes = lax.broadcasted_iota(jnp.uint32, (_IDXR, 128), 1)
    p2 = rows * jnp.uint32(128) + lanes
    thi = (p2 >= jnp.uint32(200000)).astype(jnp.uint32)
    p = p2 - thi * jnp.uint32(200000)
    bb = (p >= jnp.uint32(100000)).astype(jnp.uint32)
    q = p - bb * jnp.uint32(100000)
    n = q >> jnp.uint32(1)
    slo = q & jnp.uint32(1)
    e = (bb * jnp.uint32(_N) + n) * jnp.uint32(4) \
        + thi * jnp.uint32(2) + slo
    zero = jnp.zeros_like(e)

    def randint_val(kh, kl):
        h0, h1 = _tf_rounds(kh[0], kh[1], zero, e)
        l0, l1 = _tf_rounds(kl[0], kl[1], zero, e)
        hi = h0 ^ h1
        lo = l0 ^ l1
        y = ((hi >> jnp.uint32(16)) * jnp.uint32(10656)
             + (hi & jnp.uint32(0xFFFF)) * jnp.uint32(17296)
             + (lo >> jnp.uint32(16)) * jnp.uint32(15536)
             + (lo & jnp.uint32(0xFFFF)))
        y = (y >> jnp.uint32(24)) * jnp.uint32(27216) \
            + (y & jnp.uint32(0xFFFFFF))
        for _ in range(5):
            y = (y >> jnp.uint32(16)) * jnp.uint32(15536) \
                + (y & jnp.uint32(0xFFFF))
        y = jnp.where(y >= jnp.uint32(100000), y - jnp.uint32(100000), y)
        y = jnp.where(y >= jnp.uint32(50000), y - jnp.uint32(50000), y)
        return (y + bb * jnp.uint32(_N)).astype(jnp.int32)

    kh1, kl1 = _np_split_keys(1)
    kh2, kl2 = _np_split_keys(2)
    outj_ref[...] = randint_val(kh1, kl1)
    outk_ref[...] = randint_val(kh2, kl2)


def _idx_gen():
    sds = jax.ShapeDtypeStruct((_IDXR, 128), jnp.int32)
    return pl.pallas_call(
        _idx_body,
        grid=(1,),
        in_specs=[],
        out_specs=(pl.BlockSpec((_IDXR, 128), lambda i: (0, 0)),
                   pl.BlockSpec((_IDXR, 128), lambda i: (0, 0))),
        out_shape=(sds, sds),
    )()


def _pack_body(x_ref, out_ref):
    a = x_ref[...]
    pr = a.reshape(a.shape[0] // 2, 2, _D)
    pad = jnp.zeros((pr.shape[0], _DP - _D), dtype=a.dtype)
    out_ref[...] = jnp.concatenate(
        [pr[:, 0, :], pad, pr[:, 1, :], pad], axis=1).astype(jnp.bfloat16)


_PBLK = 2000


def _pack(x2):
    return pl.pallas_call(
        _pack_body,
        grid=(_BN // (2 * _PBLK),),
        in_specs=[pl.BlockSpec((2 * _PBLK, _D), lambda i: (i, 0))],
        out_specs=pl.BlockSpec((_PBLK, 2 * _DP), lambda i: (i, 0)),
        out_shape=jax.ShapeDtypeStruct((_BN // 2, 2 * _DP), jnp.bfloat16),
    )(x2)


def _sc_gather_body(x_hbm, idxj_hbm, idxk_hbm, out_hbm, idx_v, rows_v, sem):
    c = lax.axis_index("c")
    s = lax.axis_index("s")
    wid = s * _NC + c
    base = wid * _RW

    @pl.when(wid < _NW // 2)
    def _():
        pltpu.sync_copy(idxj_hbm.at[pl.ds(wid * _RW, _RW)], idx_v)

    @pl.when(wid >= _NW // 2)
    def _():
        pltpu.sync_copy(idxk_hbm.at[pl.ds((wid - _NW // 2) * _RW, _RW)],
                        idx_v)

    def run_block(off, sizes):
        handles = []
        pos = 0
        for sz in sizes:
            handles.append(pltpu.async_copy(
                x_hbm.at[idx_v.at[pl.ds(off + pos, sz)]],
                rows_v.at[pl.ds(pos, sz)],
                sem))
            pos += sz
        for h in handles:
            h.wait()
        pltpu.sync_copy(rows_v.at[pl.ds(0, pos)],
                        out_hbm.at[pl.ds(base + off, pos)])

    def blk(i, carry):
        run_block(i * _MB, [_SUB] * _K)
        return carry

    lax.fori_loop(0, _NFULL, blk, 0)
    run_block(_NFULL * _MB, [_SUB, _SUB, _SUB, _TAIL - 3 * _SUB])


@functools.cache
def _sc_gather():
    return pl.kernel(
        _sc_gather_body,
        out_type=jax.ShapeDtypeStruct((_R, _DP), jnp.bfloat16),
        mesh=plsc.VectorSubcoreMesh(core_axis_name="c", subcore_axis_name="s"),
        scratch_types=[
            pltpu.VMEM((_RW,), jnp.int32),
            pltpu.VMEM((_MB, _DP), jnp.bfloat16),
            pltpu.SemaphoreType.DMA,
        ],
        compiler_params=pltpu.CompilerParams(use_tc_tiling_on_sc=False),
    )


_BLK = 2000
_NB = _BN // _BLK


def _mlp_body(xlo_ref, glo_j_ref, ghi_j_ref, glo_k_ref, ghi_k_ref,
              w1i_ref, w1j_ref, w1k_ref, b1_ref, w2_ref, b2_ref, out_ref):
    a = jnp.dot(xlo_ref[...], w1i_ref[...]) + b1_ref[...]
    a2 = jnp.concatenate([a, a], axis=1)
    f32 = lambda r: r[...].astype(jnp.float32)
    hlo = a2 + jnp.dot(f32(glo_j_ref), w1j_ref[...])
    hlo = hlo + jnp.dot(f32(glo_k_ref), w1k_ref[...])
    hhi = a2 + jnp.dot(f32(ghi_j_ref), w1j_ref[...])
    hhi = hhi + jnp.dot(f32(ghi_k_ref), w1k_ref[...])
    hlo = jax.nn.gelu(hlo, approximate=True)
    hhi = jax.nn.gelu(hhi, approximate=True)
    t = (hlo[:, :_D] + hlo[:, _D:] + hhi[:, :_D] + hhi[:, _D:]) * (1.0 / _S)
    out_ref[...] = jnp.dot(t, w2_ref[...]) + b2_ref[...]


def _mlp(x2, g2, w1i, w1jb, w1kb, b1, w2, b2):
    cspec = lambda shape: pl.BlockSpec(shape, lambda nb: (0, 0))
    gspec = lambda sec: pl.BlockSpec(
        (_BLK, 2 * _DP), lambda nb, sec=sec: (sec * _NB + nb, 0))
    return pl.pallas_call(
        _mlp_body,
        grid=(_NB,),
        in_specs=[
            pl.BlockSpec((_BLK, _D), lambda nb: (nb, 0)),
            gspec(0), gspec(1), gspec(2), gspec(3),
            cspec((_D, _D)), cspec((2 * _DP, 2 * _D)),
            cspec((2 * _DP, 2 * _D)), cspec((1, _D)),
            cspec((_D, _D)), cspec((1, _D)),
        ],
        out_specs=pl.BlockSpec((_BLK, _D), lambda nb: (nb, 0)),
        out_shape=jax.ShapeDtypeStruct((_BN, _D), jnp.float32),
    )(x2, g2, g2, g2, g2, w1i, w1jb, w1kb, b1, w2, b2)


def _block_diag_w(w):
    z = jnp.zeros((_DP - _D, _D), jnp.float32)
    left = jnp.concatenate([w, z, jnp.zeros((_DP, _D), jnp.float32)], axis=0)
    right = jnp.concatenate([jnp.zeros((_DP, _D), jnp.float32), w, z], axis=0)
    return jnp.concatenate([left, right], axis=1)


def kernel(x, W1, b1, W2, b2):
    B, N, D = x.shape
    idx_j, idx_k = _idx_gen()
    x2 = x.reshape(B * N, D)
    xp = _pack(x2)
    g = _sc_gather()(xp.reshape(_BN, _DP),
                     idx_j.reshape(_Q), idx_k.reshape(_Q))
    g2 = g.reshape(_R // 2, 2 * _DP)
    out = _mlp(x2, g2, W1[:D], _block_diag_w(W1[D:2 * D]),
               _block_diag_w(W1[2 * D:]), b1.reshape(1, D), W2,
               b2.reshape(1, D))
    return out.reshape(B, N, D)


# speedup vs baseline: 1.9653x; 1.9653x over previous
import functools

import numpy as np

import jax
import jax.numpy as jnp
from jax import lax
from jax.experimental import pallas as pl
from jax.experimental.pallas import tpu as pltpu
from jax.experimental.pallas import tpu_sc as plsc

_B, _N, _D, _S = 2, 50000, 40, 4
_DP = 64
_BN = _B * _N
_Q = _S * _BN
_R = 2 * _Q

_NC, _NS = 2, 16
_NW = _NC * _NS
_RW = _R // _NW
_SUB = 128
_K = 12
_MB = _SUB * _K
_NFULL = _RW // _MB
_TAIL = _RW - _NFULL * _MB


_TF_ROTS = ((13, 15, 26, 6), (17, 29, 16, 24),
            (13, 15, 26, 6), (17, 29, 16, 24), (13, 15, 26, 6))


def _np_threefry2x32(k0, k1, x0, x1):
    with np.errstate(over="ignore"):
        ks = [np.uint32(k0), np.uint32(k1),
              np.uint32(k0) ^ np.uint32(k1) ^ np.uint32(0x1BD11BDA)]
        x = [np.uint32(x0) + ks[0], np.uint32(x1) + ks[1]]
        for i, rots in enumerate(_TF_ROTS):
            for r in rots:
                x[0] = (x[0] + x[1]).astype(np.uint32)
                x[1] = x[0] ^ ((x[1] << np.uint32(r))
                               | (x[1] >> np.uint32(32 - r)))
            x[0] = (x[0] + ks[(i + 1) % 3]).astype(np.uint32)
            x[1] = (x[1] + ks[(i + 2) % 3] + np.uint32(i + 1)).astype(
                np.uint32)
        return x[0], x[1]


def _np_split_keys(seed):
    b1, b2 = _np_threefry2x32(0, seed,
                              np.uint32([0, 0]), np.uint32([0, 1]))
    return (int(b1[0]), int(b2[0])), (int(b1[1]), int(b2[1]))


def _tf_rounds(k0, k1, x0, x1):
    ks = (jnp.uint32(k0), jnp.uint32(k1),
          jnp.uint32(np.uint32(k0) ^ np.uint32(k1) ^ np.uint32(0x1BD11BDA)))
    x0 = x0 + ks[0]
    x1 = x1 + ks[1]
    for i, rots in enumerate(_TF_ROTS):
        for r in rots:
            x0 = x0 + x1
            x1 = x0 ^ ((x1 << jnp.uint32(r)) | (x1 >> jnp.uint32(32 - r)))
        x0 = x0 + ks[(i + 1) % 3]
        x1 = x1 + ks[(i + 2) % 3] + jnp.uint32(i + 1)
    return x0, x1


_IDXR = _Q // 128


def _idx_body(outj_ref, outk_ref):
    rows = lax.broadcasted_iota(jnp.uint32, (_IDXR, 128), 0)
    lanes = lax.broadcasted_iota(jnp.uint32, (_IDXR, 128), 1)
    p2 = rows * jnp.uint32(128) + lanes
    thi = (p2 >= jnp.uint32(200000)).astype(jnp.uint32)
    p = p2 - thi * jnp.uint32(200000)
    bb = (p >= jnp.uint32(100000)).astype(jnp.uint32)
    q = p - bb * jnp.uint32(100000)
    n = q >> jnp.uint32(1)
    slo = q & jnp.uint32(1)
    e = (bb * jnp.uint32(_N) + n) * jnp.uint32(4) \
        + thi * jnp.uint32(2) + slo
    zero = jnp.zeros_like(e)

    def randint_val(kh, kl):
        h0, h1 = _tf_rounds(kh[0], kh[1], zero, e)
        l0, l1 = _tf_rounds(kl[0], kl[1], zero, e)
        hi = h0 ^ h1
        lo = l0 ^ l1
        y = ((hi >> jnp.uint32(16)) * jnp.uint32(10656)
             + (hi & jnp.uint32(0xFFFF)) * jnp.uint32(17296)
             + (lo >> jnp.uint32(16)) * jnp.uint32(15536)
             + (lo & jnp.uint32(0xFFFF)))
        y = (y >> jnp.uint32(24)) * jnp.uint32(27216) \
            + (y & jnp.uint32(0xFFFFFF))
        for _ in range(5):
            y = (y >> jnp.uint32(16)) * jnp.uint32(15536) \
                + (y & jnp.uint32(0xFFFF))
        y = jnp.where(y >= jnp.uint32(100000), y - jnp.uint32(100000), y)
        y = jnp.where(y >= jnp.uint32(50000), y - jnp.uint32(50000), y)
        return (y + bb * jnp.uint32(_N)).astype(jnp.int32)

    kh1, kl1 = _np_split_keys(1)
    kh2, kl2 = _np_split_keys(2)
    outj_ref[...] = randint_val(kh1, kl1)
    outk_ref[...] = randint_val(kh2, kl2)


def _idx_gen():
    sds = jax.ShapeDtypeStruct((_IDXR, 128), jnp.int32)
    return pl.pallas_call(
        _idx_body,
        grid=(1,),
        in_specs=[],
        out_specs=(pl.BlockSpec((_IDXR, 128), lambda i: (0, 0)),
                   pl.BlockSpec((_IDXR, 128), lambda i: (0, 0))),
        out_shape=(sds, sds),
    )()


def _pack_body(x_ref, out_ref):
    a = x_ref[...]
    pr = a.reshape(a.shape[0] // 2, 2, _D)
    pad = jnp.zeros((pr.shape[0], _DP - _D), dtype=a.dtype)
    out_ref[...] = jnp.concatenate(
        [pr[:, 0, :], pad, pr[:, 1, :], pad], axis=1)


_PBLK = 2000


def _pack(x2):
    return pl.pallas_call(
        _pack_body,
        grid=(_BN // (2 * _PBLK),),
        in_specs=[pl.BlockSpec((2 * _PBLK, _D), lambda i: (i, 0))],
        out_specs=pl.BlockSpec((_PBLK, 2 * _DP), lambda i: (i, 0)),
        out_shape=jax.ShapeDtypeStruct((_BN // 2, 2 * _DP), jnp.float32),
    )(x2)


def _sc_gather_body(x_hbm, idxj_hbm, idxk_hbm, out_hbm, idx_v, rows_v, sem):
    c = lax.axis_index("c")
    s = lax.axis_index("s")
    wid = s * _NC + c
    base = wid * _RW

    @pl.when(wid < _NW // 2)
    def _():
        pltpu.sync_copy(idxj_hbm.at[pl.ds(wid * _RW, _RW)], idx_v)

    @pl.when(wid >= _NW // 2)
    def _():
        pltpu.sync_copy(idxk_hbm.at[pl.ds((wid - _NW // 2) * _RW, _RW)],
                        idx_v)

    def run_block(off, sizes):
        handles = []
        pos = 0
        for sz in sizes:
            handles.append(pltpu.async_copy(
                x_hbm.at[idx_v.at[pl.ds(off + pos, sz)]],
                rows_v.at[pl.ds(pos, sz)],
                sem))
            pos += sz
        for h in handles:
            h.wait()
        pltpu.sync_copy(rows_v.at[pl.ds(0, pos)],
                        out_hbm.at[pl.ds(base + off, pos)])

    def blk(i, carry):
        run_block(i * _MB, [_SUB] * _K)
        return carry

    lax.fori_loop(0, _NFULL, blk, 0)
    run_block(_NFULL * _MB, [_SUB, _SUB, _SUB, _TAIL - 3 * _SUB])


@functools.cache
def _sc_gather():
    return pl.kernel(
        _sc_gather_body,
        out_type=jax.ShapeDtypeStruct((_R, _DP), jnp.float32),
        mesh=plsc.VectorSubcoreMesh(core_axis_name="c", subcore_axis_name="s"),
        scratch_types=[
            pltpu.VMEM((_RW,), jnp.int32),
            pltpu.VMEM((_MB, _DP), jnp.float32),
            pltpu.SemaphoreType.DMA,
        ],
        compiler_params=pltpu.CompilerParams(use_tc_tiling_on_sc=False),
    )


_BLK = 2000
_NB = _BN // _BLK


def _mlp_body(xlo_ref, glo_j_ref, ghi_j_ref, glo_k_ref, ghi_k_ref,
              w1i_ref, w1j_ref, w1k_ref, b1_ref, w2_ref, b2_ref, out_ref):
    a = jnp.dot(xlo_ref[...], w1i_ref[...]) + b1_ref[...]
    a2 = jnp.concatenate([a, a], axis=1)
    hlo = a2 + jnp.dot(glo_j_ref[...], w1j_ref[...])
    hlo = hlo + jnp.dot(glo_k_ref[...], w1k_ref[...])
    hhi = a2 + jnp.dot(ghi_j_ref[...], w1j_ref[...])
    hhi = hhi + jnp.dot(ghi_k_ref[...], w1k_ref[...])
    hlo = jax.nn.gelu(hlo, approximate=True)
    hhi = jax.nn.gelu(hhi, approximate=True)
    t = (hlo[:, :_D] + hlo[:, _D:] + hhi[:, :_D] + hhi[:, _D:]) * (1.0 / _S)
    out_ref[...] = jnp.dot(t, w2_ref[...]) + b2_ref[...]


def _mlp(x2, g2, w1i, w1jb, w1kb, b1, w2, b2):
    cspec = lambda shape: pl.BlockSpec(shape, lambda nb: (0, 0))
    gspec = lambda sec: pl.BlockSpec(
        (_BLK, 2 * _DP), lambda nb, sec=sec: (sec * _NB + nb, 0))
    return pl.pallas_call(
        _mlp_body,
        grid=(_NB,),
        in_specs=[
            pl.BlockSpec((_BLK, _D), lambda nb: (nb, 0)),
            gspec(0), gspec(1), gspec(2), gspec(3),
            cspec((_D, _D)), cspec((2 * _DP, 2 * _D)),
            cspec((2 * _DP, 2 * _D)), cspec((1, _D)),
            cspec((_D, _D)), cspec((1, _D)),
        ],
        out_specs=pl.BlockSpec((_BLK, _D), lambda nb: (nb, 0)),
        out_shape=jax.ShapeDtypeStruct((_BN, _D), jnp.float32),
    )(x2, g2, g2, g2, g2, w1i, w1jb, w1kb, b1, w2, b2)


def _block_diag_w(w):
    z = jnp.zeros((_DP - _D, _D), jnp.float32)
    left = jnp.concatenate([w, z, jnp.zeros((_DP, _D), jnp.float32)], axis=0)
    right = jnp.concatenate([jnp.zeros((_DP, _D), jnp.float32), w, z], axis=0)
    return jnp.concatenate([left, right], axis=1)


def kernel(x, W1, b1, W2, b2):
    B, N, D = x.shape
    idx_j, idx_k = _idx_gen()
    x2 = x.reshape(B * N, D)
    xp = _pack(x2)
    g = _sc_gather()(xp.reshape(_BN, _DP),
                     idx_j.reshape(_Q), idx_k.reshape(_Q))
    g2 = g.reshape(_R // 2, 2 * _DP)
    out = _mlp(x2, g2, W1[:D], _block_diag_w(W1[D:2 * D]),
               _block_diag_w(W1[2 * D:]), b1.reshape(1, D), W2,
               b2.reshape(1, D))
    return out.reshape(B, N, D)


# speedup vs baseline: 2.0345x; 1.0352x over previous
import functools

import numpy as np

import jax
import jax.numpy as jnp
from jax import lax
from jax.experimental import pallas as pl
from jax.experimental.pallas import tpu as pltpu
from jax.experimental.pallas import tpu_sc as plsc

_B, _N, _D, _S = 2, 50000, 40, 4
_DP = 64
_BN = _B * _N
_Q = _S * _BN
_R = 2 * _Q

_NC, _NS = 2, 16
_NW = _NC * _NS
_RW = _R // _NW
_SUB = 128
_K = 12
_MB = _SUB * _K
_NFULL = _RW // _MB
_TAIL = _RW - _NFULL * _MB


_TF_ROTS = ((13, 15, 26, 6), (17, 29, 16, 24),
            (13, 15, 26, 6), (17, 29, 16, 24), (13, 15, 26, 6))


def _np_threefry2x32(k0, k1, x0, x1):
    with np.errstate(over="ignore"):
        ks = [np.uint32(k0), np.uint32(k1),
              np.uint32(k0) ^ np.uint32(k1) ^ np.uint32(0x1BD11BDA)]
        x = [np.uint32(x0) + ks[0], np.uint32(x1) + ks[1]]
        for i, rots in enumerate(_TF_ROTS):
            for r in rots:
                x[0] = (x[0] + x[1]).astype(np.uint32)
                x[1] = x[0] ^ ((x[1] << np.uint32(r))
                               | (x[1] >> np.uint32(32 - r)))
            x[0] = (x[0] + ks[(i + 1) % 3]).astype(np.uint32)
            x[1] = (x[1] + ks[(i + 2) % 3] + np.uint32(i + 1)).astype(
                np.uint32)
        return x[0], x[1]


def _np_split_keys(seed):
    b1, b2 = _np_threefry2x32(0, seed,
                              np.uint32([0, 0]), np.uint32([0, 1]))
    return (int(b1[0]), int(b2[0])), (int(b1[1]), int(b2[1]))


def _tf_rounds(k0, k1, x0, x1):
    ks = (jnp.uint32(k0), jnp.uint32(k1),
          jnp.uint32(np.uint32(k0) ^ np.uint32(k1) ^ np.uint32(0x1BD11BDA)))
    x0 = x0 + ks[0]
    x1 = x1 + ks[1]
    for i, rots in enumerate(_TF_ROTS):
        for r in rots:
            x0 = x0 + x1
            x1 = x0 ^ ((x1 << jnp.uint32(r)) | (x1 >> jnp.uint32(32 - r)))
        x0 = x0 + ks[(i + 1) % 3]
        x1 = x1 + ks[(i + 2) % 3] + jnp.uint32(i + 1)
    return x0, x1


_IDXR = _Q // 128


def _idx_body(outj_ref, outk_ref):
    rows = lax.broadcasted_iota(jnp.uint32, (_IDXR, 128), 0)
    lanes = lax.broadcasted_iota(jnp.uint32, (_IDXR, 128), 1)
    p2 = rows * jnp.uint32(128) + lanes
    thi = (p2 >= jnp.uint32(200000)).astype(jnp.uint32)
    p = p2 - thi * jnp.uint32(200000)
    bb = (p >= jnp.uint32(100000)).astype(jnp.uint32)
    q = p - bb * jnp.uint32(100000)
    n = q >> jnp.uint32(1)
    slo = q & jnp.uint32(1)
    e = (bb * jnp.uint32(_N) + n) * jnp.uint32(4) \
        + thi * jnp.uint32(2) + slo
    zero = jnp.zeros_like(e)

    def randint_val(kh, kl):
        h0, h1 = _tf_rounds(kh[0], kh[1], zero, e)
        l0, l1 = _tf_rounds(kl[0], kl[1], zero, e)
        hi = h0 ^ h1
        lo = l0 ^ l1
        y = ((hi >> jnp.uint32(16)) * jnp.uint32(10656)
             + (hi & jnp.uint32(0xFFFF)) * jnp.uint32(17296)
             + (lo >> jnp.uint32(16)) * jnp.uint32(15536)
             + (lo & jnp.uint32(0xFFFF)))
        y = (y >> jnp.uint32(24)) * jnp.uint32(27216) \
            + (y & jnp.uint32(0xFFFFFF))
        for _ in range(5):
            y = (y >> jnp.uint32(16)) * jnp.uint32(15536) \
                + (y & jnp.uint32(0xFFFF))
        y = jnp.where(y >= jnp.uint32(100000), y - jnp.uint32(100000), y)
        y = jnp.where(y >= jnp.uint32(50000), y - jnp.uint32(50000), y)
        return (y + bb * jnp.uint32(_N)).astype(jnp.int32)

    kh1, kl1 = _np_split_keys(1)
    kh2, kl2 = _np_split_keys(2)
    outj_ref[...] = randint_val(kh1, kl1)
    outk_ref[...] = randint_val(kh2, kl2)


def _idx_gen():
    sds = jax.ShapeDtypeStruct((_IDXR, 128), jnp.int32)
    return pl.pallas_call(
        _idx_body,
        grid=(1,),
        in_specs=[],
        out_specs=(pl.BlockSpec((_IDXR, 128), lambda i: (0, 0)),
                   pl.BlockSpec((_IDXR, 128), lambda i: (0, 0))),
        out_shape=(sds, sds),
    )()


def _pack_body(x_ref, out_ref):
    a = x_ref[...]
    pr = a.reshape(a.shape[0] // 2, 2, _D)
    pad = jnp.zeros((pr.shape[0], _DP - _D), dtype=a.dtype)
    out_ref[...] = jnp.concatenate(
        [pr[:, 0, :], pad, pr[:, 1, :], pad], axis=1)


_PBLK = 2000


def _pack(x2):
    return pl.pallas_call(
        _pack_body,
        grid=(_BN // (2 * _PBLK),),
        in_specs=[pl.BlockSpec((2 * _PBLK, _D), lambda i: (i, 0))],
        out_specs=pl.BlockSpec((_PBLK, 2 * _DP), lambda i: (i, 0)),
        out_shape=jax.ShapeDtypeStruct((_BN // 2, 2 * _DP), jnp.float32),
    )(x2)


def _sc_gather_body(x_hbm, idxj_hbm, idxk_hbm, out_hbm, idx_v, rows_v, sem):
    c = lax.axis_index("c")
    s = lax.axis_index("s")
    wid = s * _NC + c
    base = wid * _RW

    @pl.when(wid < _NW // 2)
    def _():
        pltpu.sync_copy(idxj_hbm.at[pl.ds(wid * _RW, _RW)], idx_v)

    @pl.when(wid >= _NW // 2)
    def _():
        pltpu.sync_copy(idxk_hbm.at[pl.ds((wid - _NW // 2) * _RW, _RW)],
                        idx_v)

    def run_block(off, sizes):
        handles = []
        pos = 0
        for sz in sizes:
            handles.append(pltpu.async_copy(
                x_hbm.at[idx_v.at[pl.ds(off + pos, sz)]],
                rows_v.at[pl.ds(pos, sz)],
                sem))
            pos += sz
        for h in handles:
            h.wait()
        pltpu.sync_copy(rows_v.at[pl.ds(0, pos)],
                        out_hbm.at[pl.ds(base + off, pos)])

    def blk(i, carry):
        run_block(i * _MB, [_SUB] * _K)
        return carry

    lax.fori_loop(0, _NFULL, blk, 0)
    run_block(_NFULL * _MB, [_SUB, _SUB, _SUB, _TAIL - 3 * _SUB])


@functools.cache
def _sc_gather():
    return pl.kernel(
        _sc_gather_body,
        out_type=jax.ShapeDtypeStruct((_R, _DP), jnp.float32),
        mesh=plsc.VectorSubcoreMesh(core_axis_name="c", subcore_axis_name="s"),
        scratch_types=[
            pltpu.VMEM((_RW,), jnp.int32),
            pltpu.VMEM((_MB, _DP), jnp.float32),
            pltpu.SemaphoreType.DMA,
        ],
        compiler_params=pltpu.CompilerParams(use_tc_tiling_on_sc=False),
    )


_BLK = 4000
_NB = _BN // _BLK


def _mlp_body(xlo_ref, glo_j_ref, ghi_j_ref, glo_k_ref, ghi_k_ref,
              w1i_ref, w1j_ref, w1k_ref, b1_ref, w2_ref, b2_ref, out_ref):
    a = jnp.dot(xlo_ref[...], w1i_ref[...]) + b1_ref[...]
    a2 = jnp.concatenate([a, a], axis=1)
    hlo = a2 + jnp.dot(glo_j_ref[...], w1j_ref[...])
    hlo = hlo + jnp.dot(glo_k_ref[...], w1k_ref[...])
    hhi = a2 + jnp.dot(ghi_j_ref[...], w1j_ref[...])
    hhi = hhi + jnp.dot(ghi_k_ref[...], w1k_ref[...])
    hlo = jax.nn.gelu(hlo, approximate=True)
    hhi = jax.nn.gelu(hhi, approximate=True)
    t = (hlo[:, :_D] + hlo[:, _D:] + hhi[:, :_D] + hhi[:, _D:]) * (1.0 / _S)
    out_ref[...] = jnp.dot(t, w2_ref[...]) + b2_ref[...]


def _mlp(x2, g2, w1i, w1jb, w1kb, b1, w2, b2):
    cspec = lambda shape: pl.BlockSpec(shape, lambda nb: (0, 0))
    gspec = lambda sec: pl.BlockSpec(
        (_BLK, 2 * _DP), lambda nb, sec=sec: (sec * _NB + nb, 0))
    return pl.pallas_call(
        _mlp_body,
        grid=(_NB,),
        in_specs=[
            pl.BlockSpec((_BLK, _D), lambda nb: (nb, 0)),
            gspec(0), gspec(1), gspec(2), gspec(3),
            cspec((_D, _D)), cspec((2 * _DP, 2 * _D)),
            cspec((2 * _DP, 2 * _D)), cspec((1, _D)),
            cspec((_D, _D)), cspec((1, _D)),
        ],
        out_specs=pl.BlockSpec((_BLK, _D), lambda nb: (nb, 0)),
        out_shape=jax.ShapeDtypeStruct((_BN, _D), jnp.float32),
    )(x2, g2, g2, g2, g2, w1i, w1jb, w1kb, b1, w2, b2)


def _block_diag_w(w):
    z = jnp.zeros((_DP - _D, _D), jnp.float32)
    left = jnp.concatenate([w, z, jnp.zeros((_DP, _D), jnp.float32)], axis=0)
    right = jnp.concatenate([jnp.zeros((_DP, _D), jnp.float32), w, z], axis=0)
    return jnp.concatenate([left, right], axis=1)


def kernel(x, W1, b1, W2, b2):
    B, N, D = x.shape
    idx_j, idx_k = _idx_gen()
    x2 = x.reshape(B * N, D)
    xp = _pack(x2)
    g = _sc_gather()(xp.reshape(_BN, _DP),
                     idx_j.reshape(_Q), idx_k.reshape(_Q))
    g2 = g.reshape(_R // 2, 2 * _DP)
    out = _mlp(x2, g2, W1[:D], _block_diag_w(W1[D:2 * D]),
               _block_diag_w(W1[2 * D:]), b1.reshape(1, D), W2,
               b2.reshape(1, D))
    return out.reshape(B, N, D)
